# baseline (speedup 1.0000x reference)
import functools

import numpy as np
import jax
import jax.numpy as jnp
from jax import lax
from jax.experimental import pallas as pl
from jax.experimental.pallas import tpu as pltpu
from jax.experimental.pallas import tpu_sc as plsc

_VOCAB = 100000
_DIM = 64
_BATCH = 1024
_SEQ = 200
_MAX_LEN = 512

_NUM_CORES = 2
_NUM_SUBCORES = 16
_NUM_WORKERS = _NUM_CORES * _NUM_SUBCORES
_SEQ_PER_W = _BATCH // _NUM_WORKERS
_NBUF = 4


def _positional_encoding_np(max_len, d):
    pos = np.arange(max_len, dtype=np.float64)[:, None]
    i = np.arange(0, d, 2, dtype=np.float64)
    angles = pos / np.power(10000.0, i / d)
    pe = np.zeros((max_len, d), dtype=np.float64)
    pe[:, 0::2] = np.sin(angles)
    pe[:, 1::2] = np.cos(angles)
    return pe.astype(np.float32)


_PE = _positional_encoding_np(_MAX_LEN, _DIM)[:_SEQ]

_mesh = plsc.VectorSubcoreMesh(
    core_axis_name="c", subcore_axis_name="s", num_cores=_NUM_CORES
)


@functools.partial(
    pl.kernel,
    out_type=jax.ShapeDtypeStruct((_BATCH, _SEQ, 2 * _DIM), jnp.float32),
    mesh=_mesh,
    compiler_params=pltpu.CompilerParams(use_tc_tiling_on_sc=False),
    scratch_types=[
        pltpu.VMEM((_SEQ_PER_W, _SEQ), jnp.int32),
        pltpu.VMEM((_NBUF, _SEQ, _DIM), jnp.float32),
        pltpu.VMEM_SHARED((_SEQ, _DIM), jnp.float32),
        pltpu.SemaphoreType.DMA,
        pltpu.SemaphoreType.DMA((_NBUF,)),
        pltpu.SemaphoreType.DMA((_NBUF,)),
        pltpu.SemaphoreType.DMA((_NBUF,)),
    ],
)
def _emb_kernel(
    x_hbm, pe_hbm, table_hbm, out_hbm,
    idx_v, rows_v, pe_sh, idx_sem, fill_sem, gath_sem, out_sem,
):
    wid = lax.axis_index("s") * _NUM_CORES + lax.axis_index("c")
    base = wid * _SEQ_PER_W

    idx_dma = pltpu.async_copy(
        x_hbm.at[pl.ds(base, _SEQ_PER_W)], idx_v, idx_sem
    )

    @pl.when(lax.axis_index("s") == 0)
    def _():
        pltpu.sync_copy(pe_hbm, pe_sh)

    plsc.subcore_barrier()

    fill_dma = {}
    gath_dma = {}
    out_dma = {}

    def start_fill(g):
        b = g % _NBUF
        if g >= _NBUF:
            out_dma.pop(g - _NBUF).wait()
        fill_dma[g] = pltpu.async_copy(pe_sh, rows_v.at[b], fill_sem.at[b])

    def start_gather(g):
        b = g % _NBUF
        if g == 0:
            idx_dma.wait()
        fill_dma.pop(g).wait()
        gath_dma[g] = pltpu.async_copy(
            table_hbm.at[idx_v.at[g]], rows_v.at[b], gath_sem.at[b], add=True
        )

    def start_out(g):
        b = g % _NBUF
        gath_dma.pop(g).wait()
        out_dma[g] = pltpu.async_copy(
            rows_v.at[b], out_hbm.at[base + g, :, pl.ds(0, _DIM)], out_sem.at[b]
        )

    for g in range(_SEQ_PER_W + 3):
        if g < _SEQ_PER_W:
            start_fill(g)
        if 1 <= g < _SEQ_PER_W + 1:
            start_gather(g - 1)
        if g >= 3:
            start_out(g - 3)

    for g in sorted(out_dma):
        out_dma[g].wait()


def kernel(X, table):
    pe = jnp.asarray(_PE)
    out_wide = _emb_kernel(X, pe, table)
    return out_wide[:, :, :_DIM]


# speedup vs baseline: 1.0052x; 1.0052x over previous
import functools

import numpy as np
import jax
import jax.numpy as jnp
from jax import lax
from jax.experimental import pallas as pl
from jax.experimental.pallas import tpu as pltpu
from jax.experimental.pallas import tpu_sc as plsc

_VOCAB = 100000
_DIM = 64
_BATCH = 1024
_SEQ = 200
_MAX_LEN = 512

_NUM_CORES = 2
_NUM_SUBCORES = 16
_NUM_WORKERS = _NUM_CORES * _NUM_SUBCORES
_SEQ_PER_W = _BATCH // _NUM_WORKERS
_NBUF = 6


def _positional_encoding_np(max_len, d):
    pos = np.arange(max_len, dtype=np.float64)[:, None]
    i = np.arange(0, d, 2, dtype=np.float64)
    angles = pos / np.power(10000.0, i / d)
    pe = np.zeros((max_len, d), dtype=np.float64)
    pe[:, 0::2] = np.sin(angles)
    pe[:, 1::2] = np.cos(angles)
    return pe.astype(np.float32)


_PE = _positional_encoding_np(_MAX_LEN, _DIM)[:_SEQ]

_mesh = plsc.VectorSubcoreMesh(
    core_axis_name="c", subcore_axis_name="s", num_cores=_NUM_CORES
)


@functools.partial(
    pl.kernel,
    out_type=jax.ShapeDtypeStruct((_BATCH, _SEQ, 2 * _DIM), jnp.float32),
    mesh=_mesh,
    compiler_params=pltpu.CompilerParams(use_tc_tiling_on_sc=False),
    scratch_types=[
        pltpu.VMEM((_SEQ_PER_W, _SEQ), jnp.int32),
        pltpu.VMEM((_NBUF, _SEQ, _DIM), jnp.float32),
        pltpu.VMEM_SHARED((_SEQ, _DIM), jnp.float32),
        pltpu.SemaphoreType.DMA,
        pltpu.SemaphoreType.DMA((_NBUF,)),
        pltpu.SemaphoreType.DMA((_NBUF,)),
        pltpu.SemaphoreType.DMA((_NBUF,)),
    ],
)
def _emb_kernel(
    x_hbm, pe_hbm, table_hbm, out_hbm,
    idx_v, rows_v, pe_sh, idx_sem, fill_sem, gath_sem, out_sem,
):
    wid = lax.axis_index("s") * _NUM_CORES + lax.axis_index("c")
    base = wid * _SEQ_PER_W

    idx_dma = pltpu.async_copy(
        x_hbm.at[pl.ds(base, _SEQ_PER_W)], idx_v, idx_sem
    )

    @pl.when(lax.axis_index("s") == 0)
    def _():
        pltpu.sync_copy(pe_hbm, pe_sh)

    plsc.subcore_barrier()

    fill_dma = {}
    gath_dma = {}
    out_dma = {}

    def start_fill(g):
        b = g % _NBUF
        if g >= _NBUF:
            out_dma.pop(g - _NBUF).wait()
        fill_dma[g] = pltpu.async_copy(pe_sh, rows_v.at[b], fill_sem.at[b])

    def start_gather(g):
        b = g % _NBUF
        if g == 0:
            idx_dma.wait()
        fill_dma.pop(g).wait()
        gath_dma[g] = pltpu.async_copy(
            table_hbm.at[idx_v.at[g]], rows_v.at[b], gath_sem.at[b], add=True
        )

    def start_out(g):
        b = g % _NBUF
        gath_dma.pop(g).wait()
        out_dma[g] = pltpu.async_copy(
            rows_v.at[b], out_hbm.at[base + g, :, pl.ds(0, _DIM)], out_sem.at[b]
        )

    for g in range(_SEQ_PER_W + 4):
        if g < _SEQ_PER_W:
            start_fill(g)
        if 1 <= g < _SEQ_PER_W + 1:
            start_gather(g - 1)
        if g >= 4:
            start_out(g - 4)

    for g in sorted(out_dma):
        out_dma[g].wait()


def kernel(X, table):
    pe = jnp.asarray(_PE)
    out_wide = _emb_kernel(X, pe, table)
    return out_wide[:, :, :_DIM]
